# Initial kernel scaffold; baseline (speedup 1.0000x reference)
import jax
import jax.numpy as jnp
from jax.experimental import pallas as pl


def kernel(source, target):
    raise NotImplementedError("write your pallas kernel here")


# speedup vs baseline: 3.3801x; 3.3801x over previous
import jax
import jax.numpy as jnp
from jax.experimental import pallas as pl
from jax.experimental.pallas import tpu as pltpu

_N = 4096
_D = 512
_BLK = 512
_NB = _N // _BLK
_M = 2 * _N
_MUL = 2.0
_NUM = 5
_LOG2E = 1.4426950408889634


def _mmd_kernel(ci_ref, cj_ref, src_ref, tgt_ref, srcT_ref, tgtT_ref,
                out_ref, acc_ref, c_ref):
    core = pl.program_id(0)
    step = pl.program_id(1)
    n_steps = pl.num_programs(1)

    @pl.when(step == 0)
    def _prologue():
        acc_ref[...] = jnp.zeros_like(acc_ref)
        colsum = jnp.zeros((1, _D), jnp.float32)
        sqsum = jnp.zeros((1, _D), jnp.float32)
        for b in range(_NB):
            sb = src_ref[b * _BLK:(b + 1) * _BLK, :].astype(jnp.float32)
            tb = tgt_ref[b * _BLK:(b + 1) * _BLK, :].astype(jnp.float32)
            colsum += (jnp.sum(sb, axis=0, keepdims=True)
                       + jnp.sum(tb, axis=0, keepdims=True))
            sqsum += (jnp.sum(sb * sb, axis=0, keepdims=True)
                      + jnp.sum(tb * tb, axis=0, keepdims=True))
        sum_sq = jnp.sum(sqsum)
        cs2 = jnp.sum(colsum * colsum)
        sum_l2 = 2.0 * _M * sum_sq - 2.0 * cs2
        bw = sum_l2 / (_M * _M - _M) / (_MUL ** (_NUM // 2))
        beta_max = bw * (_MUL ** (_NUM - 1))
        c_ref[0] = _LOG2E / beta_max

    idx = core * n_steps + step
    i = ci_ref[idx]
    j = cj_ref[idx]
    ri = pl.multiple_of(i * _BLK, _BLK)
    rj = pl.multiple_of(j * _BLK, _BLK)

    si = src_ref[pl.ds(ri, _BLK), :]
    ti = tgt_ref[pl.ds(ri, _BLK), :]
    sjT = srcT_ref[:, pl.ds(rj, _BLK)]
    tjT = tgtT_ref[:, pl.ds(rj, _BLK)]
    sj = src_ref[pl.ds(rj, _BLK), :]
    tj = tgt_ref[pl.ds(rj, _BLK), :]

    c = c_ref[0]

    def sq_rows(a):
        af = a.astype(jnp.float32)
        return jnp.sum(af * af, axis=1, keepdims=True)

    nsi = sq_rows(si) * (-c)
    nti = sq_rows(ti) * (-c)
    nsj = sq_rows(sj).reshape(1, _BLK) * (-c)
    ntj = sq_rows(tj).reshape(1, _BLK) * (-c)

    c2 = 2.0 * c

    def ksum(a, bT, na, nb):
        g = jnp.dot(a, bT, preferred_element_type=jnp.float32)
        arg = (g * c2 + na) + nb
        t = jnp.exp2(arg)
        t2 = t * t
        t4 = t2 * t2
        t8 = t4 * t4
        t16 = t8 * t8
        return ((t + t2) + (t4 + t8)) + t16

    combo = ((ksum(si, sjT, nsi, nsj) + ksum(ti, tjT, nti, ntj))
             - (ksum(si, tjT, nsi, ntj) + ksum(ti, sjT, nti, nsj)))
    w = jnp.where(i == j, 1.0, 2.0).astype(jnp.float32)
    acc_ref[...] += w * combo

    @pl.when(step == n_steps - 1)
    def _epilogue():
        rowsum = jnp.sum(acc_ref[...], axis=1, keepdims=True)
        total = jnp.sum(rowsum, axis=0, keepdims=True)
        out_ref[...] = (total * (1.0 / (_N * _N))).reshape(1, 1, 1)


def kernel(source, target):
    src16 = source.astype(jnp.bfloat16)
    tgt16 = target.astype(jnp.bfloat16)
    srcT = src16.T
    tgtT = tgt16.T

    cells = [(i, j) for i in range(_NB) for j in range(i, _NB)]
    ci = jnp.array([c[0] for c in cells], dtype=jnp.int32)
    cj = jnp.array([c[1] for c in cells], dtype=jnp.int32)
    n_cells = len(cells)
    half = n_cells // 2

    vmem_spec = pl.BlockSpec(memory_space=pltpu.VMEM)
    out = pl.pallas_call(
        _mmd_kernel,
        out_shape=jax.ShapeDtypeStruct((2, 1, 1), jnp.float32),
        grid_spec=pltpu.PrefetchScalarGridSpec(
            num_scalar_prefetch=2,
            grid=(2, half),
            in_specs=[vmem_spec, vmem_spec, vmem_spec, vmem_spec],
            out_specs=pl.BlockSpec((1, 1, 1), lambda c, s, ci, cj: (c, 0, 0)),
            scratch_shapes=[
                pltpu.VMEM((_BLK, _BLK), jnp.float32),
                pltpu.SMEM((1,), jnp.float32),
            ],
        ),
        compiler_params=pltpu.CompilerParams(
            dimension_semantics=("parallel", "arbitrary"),
            vmem_limit_bytes=48 * 1024 * 1024,
        ),
        name="mmd_loss",
    )(ci, cj, src16, tgt16, srcT, tgtT)
    return (out[0, 0, 0] + out[1, 0, 0])


# speedup vs baseline: 3.9343x; 1.1640x over previous
import jax
import jax.numpy as jnp
from jax.experimental import pallas as pl
from jax.experimental.pallas import tpu as pltpu

_N = 4096
_D = 512
_BLK = 512
_NB = _N // _BLK
_M = 2 * _N
_MUL = 2.0
_NUM = 5
_LOG2E = 1.4426950408889634
_NT = (((1,), (1,)), ((), ()))


def _mmd_kernel(ci_ref, cj_ref, src_ref, tgt_ref,
                out_ref, acc_ref, sqc_ref, c_ref):
    step = pl.program_id(0)
    n_steps = pl.num_programs(0)

    @pl.when(step == 0)
    def _prologue():
        acc_ref[...] = jnp.zeros_like(acc_ref)
        colsum = jnp.zeros((1, _D), jnp.float32)
        sqsum = jnp.zeros((1, _D), jnp.float32)
        for b in range(_NB):
            sb = src_ref[b * _BLK:(b + 1) * _BLK, :].astype(jnp.float32)
            tb = tgt_ref[b * _BLK:(b + 1) * _BLK, :].astype(jnp.float32)
            colsum += (jnp.sum(sb, axis=0, keepdims=True)
                       + jnp.sum(tb, axis=0, keepdims=True))
            sqsum += (jnp.sum(sb * sb, axis=0, keepdims=True)
                      + jnp.sum(tb * tb, axis=0, keepdims=True))
        sum_sq = jnp.sum(sqsum)
        cs2 = jnp.sum(colsum * colsum)
        sum_l2 = 2.0 * _M * sum_sq - 2.0 * cs2
        bw = sum_l2 / (_M * _M - _M) / (_MUL ** (_NUM // 2))
        beta_max = bw * (_MUL ** (_NUM - 1))
        c = _LOG2E / beta_max
        c_ref[0] = c
        for b in range(_NB):
            sb = src_ref[b * _BLK:(b + 1) * _BLK, :].astype(jnp.float32)
            tb = tgt_ref[b * _BLK:(b + 1) * _BLK, :].astype(jnp.float32)
            sqc_ref[0, b] = (jnp.sum(sb * sb, axis=1, keepdims=True)
                             * (-c)).reshape(1, _BLK)
            sqc_ref[1, b] = (jnp.sum(tb * tb, axis=1, keepdims=True)
                             * (-c)).reshape(1, _BLK)

    i = ci_ref[step]
    j = cj_ref[step]
    ri = pl.multiple_of(i * _BLK, _BLK)
    rj = pl.multiple_of(j * _BLK, _BLK)

    si = src_ref[pl.ds(ri, _BLK), :]
    ti = tgt_ref[pl.ds(ri, _BLK), :]
    sj = src_ref[pl.ds(rj, _BLK), :]
    tj = tgt_ref[pl.ds(rj, _BLK), :]

    c = c_ref[0]

    def sq_rows(a):
        af = a.astype(jnp.float32)
        return jnp.sum(af * af, axis=1, keepdims=True)

    nsi = sq_rows(si) * (-c)
    nti = sq_rows(ti) * (-c)
    nsj = sqc_ref[0, j]
    ntj = sqc_ref[1, j]

    c2 = 2.0 * c

    def ksum(a, b, na, nb):
        g = jax.lax.dot_general(a, b, _NT, preferred_element_type=jnp.float32)
        arg = (g * c2 + na) + nb
        t = jnp.exp2(arg)
        t2 = t * t
        t4 = t2 * t2
        t8 = t4 * t4
        t16 = t8 * t8
        return ((t + t2) + (t4 + t8)) + t16

    combo = ((ksum(si, sj, nsi, nsj) + ksum(ti, tj, nti, ntj))
             - (ksum(si, tj, nsi, ntj) + ksum(ti, sj, nti, nsj)))
    w = jnp.where(i == j, 1.0, 2.0).astype(jnp.float32)
    acc_ref[...] += w * combo

    @pl.when(step == n_steps - 1)
    def _epilogue():
        rowsum = jnp.sum(acc_ref[...], axis=1, keepdims=True)
        total = jnp.sum(rowsum, axis=0, keepdims=True)
        out_ref[...] = total * (1.0 / (_N * _N))


def kernel(source, target):
    src16 = source.astype(jnp.bfloat16)
    tgt16 = target.astype(jnp.bfloat16)

    cells = [(i, j) for i in range(_NB) for j in range(i, _NB)]
    ci = jnp.array([c[0] for c in cells], dtype=jnp.int32)
    cj = jnp.array([c[1] for c in cells], dtype=jnp.int32)
    n_cells = len(cells)

    vmem_spec = pl.BlockSpec(memory_space=pltpu.VMEM)
    out = pl.pallas_call(
        _mmd_kernel,
        out_shape=jax.ShapeDtypeStruct((1, 1), jnp.float32),
        grid_spec=pltpu.PrefetchScalarGridSpec(
            num_scalar_prefetch=2,
            grid=(n_cells,),
            in_specs=[vmem_spec, vmem_spec],
            out_specs=pl.BlockSpec((1, 1), lambda s, ci, cj: (0, 0)),
            scratch_shapes=[
                pltpu.VMEM((_BLK, _BLK), jnp.float32),
                pltpu.VMEM((2, _NB, 1, _BLK), jnp.float32),
                pltpu.SMEM((1,), jnp.float32),
            ],
        ),
        compiler_params=pltpu.CompilerParams(
            dimension_semantics=("arbitrary",),
            vmem_limit_bytes=48 * 1024 * 1024,
        ),
        name="mmd_loss",
    )(ci, cj, src16, tgt16)
    return out[0, 0]
